# baseline (speedup 1.0000x reference)
import functools

import jax
import jax.numpy as jnp
from jax import lax
from jax.experimental import pallas as pl
from jax.experimental.pallas import tpu as pltpu
from jax.experimental.pallas import tpu_sc as plsc

B = 16384
D = 32
NC, NS, L = 2, 16, 16
NW = NC * NS
NPW = B // NW
NR = 2 * NPW
NG = NPW // L

_mesh = plsc.VectorSubcoreMesh(
    core_axis_name="c", subcore_axis_name="s", num_cores=NC, num_subcores=NS
)


def _rsqrt(x):
    i = plsc.bitcast(x, jnp.int32)
    i = jnp.int32(0x5F3759DF) - (i >> 1)
    y = plsc.bitcast(i, jnp.float32)
    for _ in range(3):
        y = y * (1.5 - 0.5 * x * y * y)
    return y


@functools.partial(
    pl.kernel,
    out_type=jax.ShapeDtypeStruct((B,), jnp.float32),
    mesh=_mesh,
    compiler_params=pltpu.CompilerParams(
        needs_layout_passes=False, use_tc_tiling_on_sc=False
    ),
    scratch_types=[
        pltpu.VMEM((NR,), jnp.int32),
        pltpu.VMEM((NR, D), jnp.float32),
        pltpu.VMEM((NPW,), jnp.float32),
        pltpu.SemaphoreType.DMA,
    ],
)
def _cosine_sc(x_hbm, table_hbm, out_hbm, idx_v, rows_v, out_v, sem):
    wid = lax.axis_index("s") * NC + lax.axis_index("c")
    base = wid * NPW

    pltpu.sync_copy(x_hbm.at[pl.ds(2 * base, NR)], idx_v)
    pltpu.async_copy(table_hbm.at[idx_v], rows_v, sem).wait()

    lane = lax.iota(jnp.int32, L)

    def group(g, carry):
        r1 = g * (2 * L) + 2 * lane
        r2 = r1 + 1
        dot = jnp.zeros((L,), jnp.float32)
        s1 = jnp.zeros((L,), jnp.float32)
        s2 = jnp.zeros((L,), jnp.float32)
        for d in range(D):
            c = jnp.full((L,), d, jnp.int32)
            e1 = plsc.load_gather(rows_v, [r1, c])
            e2 = plsc.load_gather(rows_v, [r2, c])
            dot = dot + e1 * e2
            s1 = s1 + e1 * e1
            s2 = s2 + e2 * e2
        sim = dot * _rsqrt(jnp.maximum(s1, 1e-16)) * _rsqrt(jnp.maximum(s2, 1e-16))
        out_v[pl.ds(g * L, L)] = 0.5 + 0.5 * sim
        return carry

    lax.fori_loop(0, NG, group, 0)

    pltpu.sync_copy(out_v, out_hbm.at[pl.ds(base, NPW)])


def kernel(x, table):
    return _cosine_sc(x.reshape(-1).astype(jnp.int32), table)
